# baseline (speedup 1.0000x reference)
import functools

import jax
import jax.numpy as jnp
from jax import lax
from jax.experimental import pallas as pl
from jax.experimental.pallas import tpu as pltpu
from jax.experimental.pallas import tpu_sc as plsc

B = 1024
F = 26
V = 100000
D = 16

NC = 2
NS = 16
L = 16
FPC = F // NC
NW = NC * NS
BPW = B // NW
NQ = B // L


def _sweep_body(tab_hbm, idx_hbm, out_hbm, rb0, rb1, idxv, accsq, sem):
    c = lax.axis_index("c")
    t = lax.axis_index("s")
    pltpu.sync_copy(idx_hbm.at[pl.ds(c * L, L), :], idxv)
    zeros = jnp.zeros((L,), jnp.float32)
    for q in range(NQ):
        accsq[pl.ds(q * L, L)] = zeros
        accsq[pl.ds(B + q * L, L)] = zeros

    H0 = 49920

    def per_field(i, chk):
        dep = lax.shift_right_logical(
            lax.convert_element_type(jnp.abs(chk[0]), jnp.int32), 31
        )
        gf = c * FPC + i
        r = gf * D + t + dep
        cp0 = pltpu.async_copy(tab_hbm.at[r // 8, r % 8, pl.ds(0, H0)], rb0, sem)
        cp1 = pltpu.async_copy(tab_hbm.at[r // 8, r % 8, pl.ds(H0, V - H0)], rb1, sem)
        cp0.wait()
        cp1.wait()
        for q in range(NQ):
            vv = idxv[i, pl.ds(q * L, L)]
            m0 = vv < H0
            v0 = plsc.load_gather(rb0, [jnp.where(m0, vv, 0)], mask=m0)
            v1 = plsc.load_gather(rb1, [jnp.where(m0, 0, vv - H0)], mask=~m0)
            val = jnp.where(m0, v0, v1)
            chk = chk + val
            a = accsq[pl.ds(q * L, L)]
            accsq[pl.ds(q * L, L)] = a + val
            s = accsq[pl.ds(B + q * L, L)]
            accsq[pl.ds(B + q * L, L)] = s + val * val
        return chk

    lax.fori_loop(0, FPC, per_field, jnp.zeros((L,), jnp.float32))
    pltpu.sync_copy(accsq, out_hbm.at[c, t, :])


def _head_body(part_hbm, par_hbm, out_hbm, pbuf, parv, outv):
    wid = lax.axis_index("s") * NC + lax.axis_index("c")
    base = wid * BPW
    blk = pl.multiple_of((wid // 4) * 128, 128)
    off = (wid % 4) * BPW
    pltpu.sync_copy(par_hbm, parv)
    for c in range(NC):
        for k in range(2):
            pltpu.sync_copy(
                part_hbm.at[c, :, pl.ds(k * B + blk, 128)], pbuf.at[c * 2 + k]
            )
    wo = parv[0, pl.ds(0, L)]
    bo = parv[1, pl.ds(0, L)]
    for g in range(BPW // L):
        tot = jnp.zeros((L,), jnp.float32)
        for d in range(D):
            a = pbuf[0, d, pl.ds(off + g * L, L)] + pbuf[2, d, pl.ds(off + g * L, L)]
            q = pbuf[1, d, pl.ds(off + g * L, L)] + pbuf[3, d, pl.ds(off + g * L, L)]
            tot = tot + (a * a - q)
        z = (0.5 * tot) * wo + bo
        outv[pl.ds(g * L, L)] = 1.0 / (1.0 + jnp.exp(-z))
    pltpu.sync_copy(outv, out_hbm.at[pl.ds(base, BPW)])


@jax.jit
def _afm_call(tab3, idx32, par):
    mesh = plsc.VectorSubcoreMesh(
        core_axis_name="c", subcore_axis_name="s", num_cores=NC, num_subcores=NS
    )
    sweep = functools.partial(
        pl.kernel,
        out_type=jax.ShapeDtypeStruct((NC, D, 2 * B), jnp.float32),
        mesh=mesh,
        compiler_params=pltpu.CompilerParams(needs_layout_passes=False),
        scratch_types=[
            pltpu.VMEM((49920,), jnp.float32),
            pltpu.VMEM((V - 49920,), jnp.float32),
            pltpu.VMEM((L, B), jnp.int32),
            pltpu.VMEM((2 * B,), jnp.float32),
            pltpu.SemaphoreType.DMA,
        ],
    )(_sweep_body)
    partials = sweep(tab3, idx32)

    head = functools.partial(
        pl.kernel,
        out_type=jax.ShapeDtypeStruct((B,), jnp.float32),
        mesh=mesh,
        compiler_params=pltpu.CompilerParams(needs_layout_passes=False),
        scratch_types=[
            pltpu.VMEM((4, D, 128), jnp.float32),
            pltpu.VMEM((2, 128), jnp.float32),
            pltpu.VMEM((BPW,), jnp.float32),
        ],
    )(_head_body)
    out = head(partials, par)
    return out


def kernel(inputs, tables, W1, b1, W2, b2, Wo, bo):
    sparse = inputs[:, 13:]
    spT = sparse.T
    pad = jnp.zeros((NC * L - F, B), jnp.int32)
    idx32 = jnp.concatenate(
        [spT[:FPC], pad[: L - FPC], spT[FPC:], pad[L - FPC :]], axis=0
    )
    tab3 = tables.transpose(0, 2, 1).reshape(F * D // 8, 8, V)
    par = jnp.stack(
        [jnp.full((128,), Wo[0, 0], jnp.float32), jnp.full((128,), bo[0], jnp.float32)]
    )
    out = _afm_call(tab3, idx32, par)
    return out.reshape(B, 1)


# speedup vs baseline: 1.0292x; 1.0292x over previous
import functools

import jax
import jax.numpy as jnp
from jax import lax
from jax.experimental import pallas as pl
from jax.experimental.pallas import tpu as pltpu
from jax.experimental.pallas import tpu_sc as plsc

B = 1024
F = 26
V = 100000
D = 16

NC = 2
NS = 16
L = 16
FPC = F // NC
NW = NC * NS
BPW = B // NW
NQ = B // L


def _sweep_body(tab_hbm, idx_hbm, out_hbm, rowbuf, idxv, accsq, sem):
    c = lax.axis_index("c")
    t = lax.axis_index("s")
    pltpu.sync_copy(idx_hbm.at[pl.ds(c * L, L), :], idxv)
    zeros = jnp.zeros((L,), jnp.float32)
    for q in range(NQ):
        accsq[pl.ds(q * L, L)] = zeros
        accsq[pl.ds(B + q * L, L)] = zeros

    def per_field(i, chk):
        dep = lax.shift_right_logical(
            lax.convert_element_type(jnp.abs(chk[0]), jnp.int32), 31
        )
        gf = c * FPC + i
        r = gf * D + t + dep
        pltpu.async_copy(tab_hbm.at[r // 8, r % 8, :], rowbuf, sem).wait()
        for q in range(NQ):
            vv = idxv[i, pl.ds(q * L, L)]
            val = plsc.load_gather(rowbuf, [vv])
            chk = chk + val
            a = accsq[pl.ds(q * L, L)]
            accsq[pl.ds(q * L, L)] = a + val
            s = accsq[pl.ds(B + q * L, L)]
            accsq[pl.ds(B + q * L, L)] = s + val * val
        return chk

    lax.fori_loop(0, FPC, per_field, jnp.zeros((L,), jnp.float32))
    pltpu.sync_copy(accsq, out_hbm.at[c, t, :])


def _head_body(part_hbm, par_hbm, out_hbm, pbuf, parv, outv):
    wid = lax.axis_index("s") * NC + lax.axis_index("c")
    base = wid * BPW
    blk = pl.multiple_of((wid // 4) * 128, 128)
    off = (wid % 4) * BPW
    pltpu.sync_copy(par_hbm, parv)
    for c in range(NC):
        for k in range(2):
            pltpu.sync_copy(
                part_hbm.at[c, :, pl.ds(k * B + blk, 128)], pbuf.at[c * 2 + k]
            )
    wo = parv[0, pl.ds(0, L)]
    bo = parv[1, pl.ds(0, L)]
    for g in range(BPW // L):
        tot = jnp.zeros((L,), jnp.float32)
        for d in range(D):
            a = pbuf[0, d, pl.ds(off + g * L, L)] + pbuf[2, d, pl.ds(off + g * L, L)]
            q = pbuf[1, d, pl.ds(off + g * L, L)] + pbuf[3, d, pl.ds(off + g * L, L)]
            tot = tot + (a * a - q)
        z = (0.5 * tot) * wo + bo
        outv[pl.ds(g * L, L)] = 1.0 / (1.0 + jnp.exp(-z))
    pltpu.sync_copy(outv, out_hbm.at[pl.ds(base, BPW)])


@jax.jit
def _afm_call(tab3, idx32, par):
    mesh = plsc.VectorSubcoreMesh(
        core_axis_name="c", subcore_axis_name="s", num_cores=NC, num_subcores=NS
    )
    sweep = functools.partial(
        pl.kernel,
        out_type=jax.ShapeDtypeStruct((NC, D, 2 * B), jnp.float32),
        mesh=mesh,
        compiler_params=pltpu.CompilerParams(needs_layout_passes=False),
        scratch_types=[
            pltpu.VMEM((V,), jnp.float32),
            pltpu.VMEM((L, B), jnp.int32),
            pltpu.VMEM((2 * B,), jnp.float32),
            pltpu.SemaphoreType.DMA,
        ],
    )(_sweep_body)
    partials = sweep(tab3, idx32)

    head = functools.partial(
        pl.kernel,
        out_type=jax.ShapeDtypeStruct((B,), jnp.float32),
        mesh=mesh,
        compiler_params=pltpu.CompilerParams(needs_layout_passes=False),
        scratch_types=[
            pltpu.VMEM((4, D, 128), jnp.float32),
            pltpu.VMEM((2, 128), jnp.float32),
            pltpu.VMEM((BPW,), jnp.float32),
        ],
    )(_head_body)
    out = head(partials, par)
    return out


def kernel(inputs, tables, W1, b1, W2, b2, Wo, bo):
    sparse = inputs[:, 13:]
    spT = sparse.T
    pad = jnp.zeros((NC * L - F, B), jnp.int32)
    idx32 = jnp.concatenate(
        [spT[:FPC], pad[: L - FPC], spT[FPC:], pad[L - FPC :]], axis=0
    )
    tab3 = tables.transpose(0, 2, 1).reshape(F * D // 8, 8, V)
    par = jnp.stack(
        [jnp.full((128,), Wo[0, 0], jnp.float32), jnp.full((128,), bo[0], jnp.float32)]
    )
    out = _afm_call(tab3, idx32, par)
    return out.reshape(B, 1)
